# Initial kernel scaffold; baseline (speedup 1.0000x reference)
import jax
import jax.numpy as jnp
from jax.experimental import pallas as pl


def kernel(prefix, embedding):
    raise NotImplementedError("write your pallas kernel here")


# speedup vs baseline: 1.3225x; 1.3225x over previous
import functools

import jax
import jax.numpy as jnp
from jax import lax
from jax.experimental import pallas as pl
from jax.experimental.pallas import tpu as pltpu
from jax.experimental.pallas import tpu_sc as plsc

NUM_TOKENS = 1000
ROW_DIM = 18432
BATCH = 1024
PRE_SEQ_LEN = 20

B = BATCH * PRE_SEQ_LEN
NC, NS = 2, 16
NW = NC * NS
BPW = B // NW
K = 4
NCHUNK = BPW // K

_mesh = plsc.VectorSubcoreMesh(core_axis_name="c", subcore_axis_name="s")


@functools.partial(
    pl.kernel,
    out_type=jax.ShapeDtypeStruct((B, ROW_DIM), jnp.float32),
    mesh=_mesh,
    scratch_types=[
        pltpu.VMEM((NCHUNK, K), jnp.int32),
        pltpu.VMEM((K, ROW_DIM), jnp.float32),
        pltpu.SemaphoreType.DMA,
    ],
)
def _gather_rows(idx_hbm, table_hbm, out_hbm, idx_v, rows_v, sem):
    wid = lax.axis_index("s") * NC + lax.axis_index("c")
    base = wid * BPW
    pltpu.sync_copy(idx_hbm.at[wid], idx_v)

    def body(g, carry):
        pltpu.async_copy(table_hbm.at[idx_v.at[g]], rows_v, sem).wait()
        pltpu.sync_copy(rows_v, out_hbm.at[pl.ds(base + g * K, K)])
        return carry

    lax.fori_loop(0, NCHUNK, body, 0)


def kernel(prefix, embedding):
    idx = prefix.reshape(NW, NCHUNK, K).astype(jnp.int32)
    out = _gather_rows(idx, embedding)
    return out.reshape(prefix.shape[0], prefix.shape[1], embedding.shape[1])


# speedup vs baseline: 1.3546x; 1.0243x over previous
import functools

import jax
import jax.numpy as jnp
from jax import lax
from jax.experimental import pallas as pl
from jax.experimental.pallas import tpu as pltpu
from jax.experimental.pallas import tpu_sc as plsc

NUM_TOKENS = 1000
ROW_DIM = 18432
BATCH = 1024
PRE_SEQ_LEN = 20

B = BATCH * PRE_SEQ_LEN
NC, NS = 2, 16
NW = NC * NS
BPW = B // NW
K = 2
NCHUNK = BPW // K
NBUF = 2
NOUTER = NCHUNK // NBUF

_mesh = plsc.VectorSubcoreMesh(core_axis_name="c", subcore_axis_name="s")


@functools.partial(
    pl.kernel,
    out_type=jax.ShapeDtypeStruct((B, ROW_DIM), jnp.float32),
    mesh=_mesh,
    scratch_types=[
        pltpu.VMEM((NCHUNK, K), jnp.int32),
        [pltpu.VMEM((K, ROW_DIM), jnp.float32) for _ in range(NBUF)],
        [pltpu.SemaphoreType.DMA for _ in range(NBUF)],
    ],
)
def _gather_rows(idx_hbm, table_hbm, out_hbm, idx_v, bufs, sems):
    wid = lax.axis_index("s") * NC + lax.axis_index("c")
    base = wid * BPW
    pltpu.sync_copy(idx_hbm.at[wid], idx_v)

    for b in range(NBUF):
        pltpu.async_copy(table_hbm.at[idx_v.at[b]], bufs[b], sems[b])

    def body(go, carry):
        for b in range(NBUF):
            g = go * NBUF + b
            pltpu.make_async_copy(table_hbm.at[idx_v.at[g]], bufs[b],
                                  sems[b]).wait()
            pltpu.sync_copy(bufs[b], out_hbm.at[pl.ds(base + g * K, K)])

            @pl.when(g + NBUF < NCHUNK)
            def _():
                pltpu.async_copy(table_hbm.at[idx_v.at[g + NBUF]], bufs[b],
                                 sems[b])
        return carry

    lax.fori_loop(0, NOUTER, body, 0)


def kernel(prefix, embedding):
    idx = prefix.reshape(NW, NCHUNK, K).astype(jnp.int32)
    out = _gather_rows(idx, embedding)
    return out.reshape(prefix.shape[0], prefix.shape[1], embedding.shape[1])


# speedup vs baseline: 1.9171x; 1.4152x over previous
import functools

import jax
import jax.numpy as jnp
from jax import lax
from jax.experimental import pallas as pl
from jax.experimental.pallas import tpu as pltpu
from jax.experimental.pallas import tpu_sc as plsc

NUM_TOKENS = 1000
ROW_DIM = 18432
BATCH = 1024
PRE_SEQ_LEN = 20

B = BATCH * PRE_SEQ_LEN
NC, NS = 2, 16
NW = NC * NS
BPW = B // NW
K = 2
NCHUNK = BPW // K
CPB = PRE_SEQ_LEN // K
BATCH_PW = BATCH // NW
NBUF = 2
NOUTER = NCHUNK // NBUF

_mesh = plsc.VectorSubcoreMesh(core_axis_name="c", subcore_axis_name="s")


@functools.partial(
    pl.kernel,
    out_type=jax.ShapeDtypeStruct((BATCH, PRE_SEQ_LEN, ROW_DIM), jnp.float32),
    mesh=_mesh,
    scratch_types=[
        pltpu.VMEM((NCHUNK, K), jnp.int32),
        [pltpu.VMEM((K, ROW_DIM), jnp.float32) for _ in range(NBUF)],
        [pltpu.SemaphoreType.DMA for _ in range(NBUF)],
    ],
)
def _gather_rows(idx_hbm, table_hbm, out_hbm, idx_v, bufs, sems):
    wid = lax.axis_index("s") * NC + lax.axis_index("c")
    bbase = wid * BATCH_PW
    pltpu.sync_copy(idx_hbm.at[wid], idx_v)

    for b in range(NBUF):
        pltpu.async_copy(table_hbm.at[idx_v.at[b]], bufs[b], sems[b])

    def body(go, carry):
        for b in range(NBUF):
            g = go * NBUF + b
            pltpu.make_async_copy(table_hbm.at[idx_v.at[g]], bufs[b],
                                  sems[b]).wait()
            bb = bbase + g // CPB
            s0 = (g % CPB) * K
            pltpu.sync_copy(bufs[b], out_hbm.at[bb, pl.ds(s0, K)])

            @pl.when(g + NBUF < NCHUNK)
            def _():
                pltpu.async_copy(table_hbm.at[idx_v.at[g + NBUF]], bufs[b],
                                 sems[b])
        return carry

    lax.fori_loop(0, NOUTER, body, 0)


def kernel(prefix, embedding):
    idx = prefix.reshape(NW, NCHUNK, K).astype(jnp.int32)
    return _gather_rows(idx, embedding)


# speedup vs baseline: 4.0375x; 2.1060x over previous
import functools

import jax
import jax.numpy as jnp
from jax import lax
from jax.experimental import pallas as pl
from jax.experimental.pallas import tpu as pltpu
from jax.experimental.pallas import tpu_sc as plsc

NUM_TOKENS = 1000
ROW_DIM = 18432
BATCH = 1024
PRE_SEQ_LEN = 20

B = BATCH * PRE_SEQ_LEN
NC, NS = 2, 16
NW = NC * NS
BPW = B // NW
K = 2
NCHUNK = BPW // K
NBUF = 2
NOUTER = NCHUNK // NBUF

_mesh = plsc.VectorSubcoreMesh(core_axis_name="c", subcore_axis_name="s")


@functools.partial(
    pl.kernel,
    out_type=jax.ShapeDtypeStruct((PRE_SEQ_LEN, BATCH, ROW_DIM), jnp.float32),
    mesh=_mesh,
    scratch_types=[
        pltpu.VMEM((NCHUNK, K), jnp.int32),
        [pltpu.VMEM((K, ROW_DIM), jnp.float32) for _ in range(NBUF)],
        [pltpu.SemaphoreType.DMA for _ in range(NBUF)],
    ],
)
def _gather_rows(idx_hbm, table_hbm, out_hbm, idx_v, bufs, sems):
    wid = lax.axis_index("s") * NC + lax.axis_index("c")
    rbase = wid * BPW
    pltpu.sync_copy(idx_hbm.at[wid], idx_v)

    for b in range(NBUF):
        pltpu.async_copy(table_hbm.at[idx_v.at[b]], bufs[b], sems[b])

    def body(go, carry):
        for b in range(NBUF):
            g = go * NBUF + b
            pltpu.make_async_copy(table_hbm.at[idx_v.at[g]], bufs[b],
                                  sems[b]).wait()
            r = rbase + g * K
            pltpu.sync_copy(bufs[b], out_hbm.at[r // BATCH,
                                                pl.ds(r % BATCH, K)])

            @pl.when(g + NBUF < NCHUNK)
            def _():
                pltpu.async_copy(table_hbm.at[idx_v.at[g + NBUF]], bufs[b],
                                 sems[b])
        return carry

    lax.fori_loop(0, NOUTER, body, 0)


def kernel(prefix, embedding):
    idx = prefix.T.reshape(NW, NCHUNK, K).astype(jnp.int32)
    out = _gather_rows(idx, embedding)
    return out.transpose(1, 0, 2)


# speedup vs baseline: 7.0808x; 1.7538x over previous
import functools

import jax
import jax.numpy as jnp
from jax import lax
from jax.experimental import pallas as pl
from jax.experimental.pallas import tpu as pltpu
from jax.experimental.pallas import tpu_sc as plsc

NUM_TOKENS = 1000
ROW_DIM = 18432
BATCH = 1024
PRE_SEQ_LEN = 20

B = BATCH * PRE_SEQ_LEN
NC, NS = 2, 16
NW = NC * NS
BPW = B // NW
GL = 16
NG = BPW // GL

_mesh = plsc.VectorSubcoreMesh(core_axis_name="c", subcore_axis_name="s")


@functools.partial(
    pl.kernel,
    out_type=jax.ShapeDtypeStruct((B, ROW_DIM), jnp.float32),
    mesh=_mesh,
    scratch_types=[
        pltpu.VMEM((BPW, 1), jnp.int32),
        pltpu.VMEM((BPW,), jnp.int32),
        pltpu.VMEM((BPW,), jnp.int32),
        pltpu.VMEM((1, ROW_DIM), jnp.float32),
        pltpu.SemaphoreType.DMA,
    ],
)
def _gather_rows(vals2_hbm, vals1_hbm, dsts_hbm, table_hbm, out_hbm,
                 vals2_v, vals1_v, dsts_v, row_v, sem):
    wid = lax.axis_index("s") * NC + lax.axis_index("c")
    pltpu.sync_copy(vals2_hbm.at[wid], vals2_v)
    pltpu.sync_copy(vals1_hbm.at[wid], vals1_v)
    pltpu.sync_copy(dsts_hbm.at[wid], dsts_v)

    def group(g, vprev):
        vv = vals1_v[pl.ds(g * GL, GL)]
        dd = dsts_v[pl.ds(g * GL, GL)]
        for k in range(GL):
            v = jnp.squeeze(lax.slice(vv, (k,), (k + 1,)))
            d = jnp.squeeze(lax.slice(dd, (k,), (k + 1,)))

            @pl.when(v != vprev)
            def _():
                pltpu.async_copy(table_hbm.at[vals2_v.at[g * GL + k]],
                                 row_v, sem).wait()

            pltpu.sync_copy(row_v, out_hbm.at[pl.ds(d, 1)])
            vprev = v
        return vprev

    lax.fori_loop(0, NG, group, jnp.int32(-1))


def kernel(prefix, embedding):
    flat = prefix.T.reshape(-1).astype(jnp.int32)
    keys = (flat << 15) | jax.lax.iota(jnp.int32, B)
    skeys = jnp.sort(keys)
    vals = skeys >> 15
    dsts = skeys & 0x7FFF
    out = _gather_rows(
        vals.reshape(NW, BPW, 1),
        vals.reshape(NW, BPW),
        dsts.reshape(NW, BPW),
        embedding,
    )
    return (out.reshape(PRE_SEQ_LEN, BATCH, ROW_DIM).transpose(1, 0, 2))
